# Initial kernel scaffold; baseline (speedup 1.0000x reference)
import functools

import jax
import jax.numpy as jnp
from jax import lax
from jax.experimental import pallas as pl
from jax.experimental.pallas import tpu as pltpu
from jax.experimental.pallas import tpu_sc as plsc

_ROWS = 16384
_COLS = 50
_B = _ROWS * _COLS
_D = 64
_NC, _NS = 2, 16
_NW = _NC * _NS
_BPW = _B // _NW
_CH = 128
_NCHUNK = _BPW // _CH
_NBUF = 2

_mesh = plsc.VectorSubcoreMesh(core_axis_name="c", subcore_axis_name="s")


@functools.partial(
    pl.kernel,
    out_type=jax.ShapeDtypeStruct((_B, _D), jnp.float32),
    mesh=_mesh,
    scratch_types=[
        pltpu.VMEM((_NCHUNK, _CH), jnp.int32),
        pltpu.VMEM((_NBUF, _CH, _D), jnp.float32),
        pltpu.SemaphoreType.DMA,
        pltpu.SemaphoreType.DMA,
    ],
)
def _emb_gather(idx_hbm, table_hbm, out_hbm, idx_v, rows_v, sem0, sem1):
    wid = lax.axis_index("s") * _NC + lax.axis_index("c")
    base = wid * _BPW
    sems = (sem0, sem1)

    pltpu.sync_copy(
        idx_hbm.at[pl.ds(base, _BPW)],
        idx_v.reshape(_BPW),
    )

    def gather_start(g, b):
        pltpu.make_async_copy(
            table_hbm.at[idx_v.at[g]], rows_v.at[b], sems[b]
        ).start()

    def gather_wait_flush(g, b):
        pltpu.make_async_copy(
            table_hbm.at[idx_v.at[g]], rows_v.at[b], sems[b]
        ).wait()
        pltpu.sync_copy(rows_v.at[b], out_hbm.at[pl.ds(base + g * _CH, _CH)])

    for b in range(_NBUF):
        gather_start(b, b)

    @pl.loop(0, _NCHUNK - _NBUF, step=_NBUF)
    def _(g):
        for b in range(_NBUF):
            gather_wait_flush(g + b, b)
            gather_start(g + b + _NBUF, b)

    for b in range(_NBUF):
        gather_wait_flush(_NCHUNK - _NBUF + b, b)


def kernel(x, table):
    flat = x.reshape(_B)
    out = _emb_gather(flat, table)
    return out.reshape(_ROWS, _COLS, _D)


# speedup vs baseline: 1.8513x; 1.8513x over previous
import functools

import jax
import jax.numpy as jnp
from jax import lax
from jax.experimental import pallas as pl
from jax.experimental.pallas import tpu as pltpu
from jax.experimental.pallas import tpu_sc as plsc

_ROWS = 16384
_COLS = 50
_B = _ROWS * _COLS
_D = 64
_NC, _NS = 2, 16
_NW = _NC * _NS
_BPW = _B // _NW
_CH = 128
_NCHUNK = _BPW // _CH
_NBUF = 2

_mesh = plsc.VectorSubcoreMesh(core_axis_name="c", subcore_axis_name="s")


@functools.partial(
    pl.kernel,
    out_type=jax.ShapeDtypeStruct((_B, _D), jnp.float32),
    mesh=_mesh,
    scratch_types=[
        pltpu.VMEM((_NCHUNK, _CH), jnp.int32),
        pltpu.VMEM((_NBUF, _CH, _D), jnp.float32),
        pltpu.SemaphoreType.DMA,
        pltpu.SemaphoreType.DMA,
    ],
    compiler_params=pltpu.CompilerParams(use_tc_tiling_on_sc=False),
)
def _emb_gather(idx_hbm, table_hbm, out_hbm, idx_v, rows_v, sem0, sem1):
    wid = lax.axis_index("s") * _NC + lax.axis_index("c")
    base = wid * _BPW
    sems = (sem0, sem1)

    pltpu.sync_copy(
        idx_hbm.at[pl.ds(wid * _NCHUNK, _NCHUNK)],
        idx_v,
    )

    def gather_start(g, b):
        pltpu.make_async_copy(
            table_hbm.at[idx_v.at[g]], rows_v.at[b], sems[b]
        ).start()

    def gather_wait_flush(g, b):
        pltpu.make_async_copy(
            table_hbm.at[idx_v.at[g]], rows_v.at[b], sems[b]
        ).wait()
        pltpu.sync_copy(rows_v.at[b], out_hbm.at[pl.ds(base + g * _CH, _CH)])

    for b in range(_NBUF):
        gather_start(b, b)

    @pl.loop(0, _NCHUNK - _NBUF, step=_NBUF)
    def _(g):
        for b in range(_NBUF):
            gather_wait_flush(g + b, b)
            gather_start(g + b + _NBUF, b)

    for b in range(_NBUF):
        gather_wait_flush(_NCHUNK - _NBUF + b, b)


def kernel(x, table):
    idx2d = x.reshape(_NW * _NCHUNK, _CH)
    out = _emb_gather(idx2d, table)
    return out.reshape(_ROWS, _COLS, _D)


# speedup vs baseline: 1.8721x; 1.0112x over previous
import functools

import jax
import jax.numpy as jnp
from jax import lax
from jax.experimental import pallas as pl
from jax.experimental.pallas import tpu as pltpu
from jax.experimental.pallas import tpu_sc as plsc

_ROWS = 16384
_COLS = 50
_B = _ROWS * _COLS
_D = 64
_NC, _NS = 2, 16
_NW = _NC * _NS
_BPW = _B // _NW
_CH = 128
_NCHUNK = _BPW // _CH
_NBUF = 4

_mesh = plsc.VectorSubcoreMesh(core_axis_name="c", subcore_axis_name="s")


@functools.partial(
    pl.kernel,
    out_type=jax.ShapeDtypeStruct((_B, _D), jnp.float32),
    mesh=_mesh,
    scratch_types=[
        pltpu.VMEM((_NCHUNK, _CH), jnp.int32),
        pltpu.VMEM((_NBUF, _CH, _D), jnp.float32),
        [pltpu.SemaphoreType.DMA] * _NBUF,
        [pltpu.SemaphoreType.DMA] * _NBUF,
    ],
    compiler_params=pltpu.CompilerParams(use_tc_tiling_on_sc=False),
)
def _emb_gather(idx_hbm, table_hbm, out_hbm, idx_v, rows_v, gsems, fsems):
    wid = lax.axis_index("s") * _NC + lax.axis_index("c")
    base = wid * _BPW

    pltpu.sync_copy(
        idx_hbm.at[pl.ds(wid * _NCHUNK, _NCHUNK)],
        idx_v,
    )

    def gather_copy(g, b):
        return pltpu.make_async_copy(
            table_hbm.at[idx_v.at[g]], rows_v.at[b], gsems[b]
        )

    def flush_copy(g, b):
        return pltpu.make_async_copy(
            rows_v.at[b], out_hbm.at[pl.ds(base + g * _CH, _CH)], fsems[b]
        )

    for b in range(_NBUF):
        gather_copy(b, b).start()

    @pl.loop(0, _NCHUNK, step=_NBUF)
    def _(g):
        for b in range(_NBUF):
            gather_copy(g + b, b).wait()
            flush_copy(g + b, b).start()
        for b in range(_NBUF):
            flush_copy(g + b, b).wait()

            @pl.when(g + _NBUF + b < _NCHUNK)
            def _():
                gather_copy(g + _NBUF + b, b).start()


def kernel(x, table):
    idx2d = x.reshape(_NW * _NCHUNK, _CH)
    out = _emb_gather(idx2d, table)
    return out.reshape(_ROWS, _COLS, _D)
